# baseline (speedup 1.0000x reference)
import functools

import jax
import jax.numpy as jnp
from jax import lax
from jax.experimental import pallas as pl
from jax.experimental.pallas import tpu as pltpu
from jax.experimental.pallas import tpu_sc as plsc

N = 10000
D = 128
E = 320000
NP = 10240
K = 128
NW = 32
CH = 80
EPW = E // NW
PPW = CH * K - EPW
NSUB = 16
RPT = NP // NSUB
HK = K // 2

@functools.cache
def _get_sc_agg():
    mesh = plsc.VectorSubcoreMesh(core_axis_name="c", subcore_axis_name="s",
                                  num_cores=2, num_subcores=NSUB)

    @functools.partial(
        pl.kernel,
        out_type=jax.ShapeDtypeStruct((2, NP, D), jnp.float32),
        mesh=mesh,
        scratch_types=[
            pltpu.VMEM((CH // 2, K), jnp.int32),
            pltpu.VMEM((CH // 2, K), jnp.int32),
            pltpu.VMEM((4 * HK, D), jnp.float32),
            pltpu.VMEM_SHARED((NP, D), jnp.float32),
            pltpu.SemaphoreType.DMA,
            pltpu.SemaphoreType.DMA,
            pltpu.SemaphoreType.DMA,
            pltpu.SemaphoreType.DMA,
        ],
    )
    def _sc_agg(h_hbm, src_hbm, dst_hbm, z_hbm, out_hbm, src_v, dst_v,
                qb, acc_sh, s0, s1, s2, s3):
        cid = lax.axis_index("c")
        sid = lax.axis_index("s")
        wid = sid * 2 + cid
        hh = CH // 2
        sems = (s0, s1, s2, s3)

        def issue(j, p):
            for half_k in range(2):
                q = 2 * p + half_k
                pltpu.async_copy(
                    h_hbm.at[src_v.at[j, pl.ds(half_k * HK, HK)]],
                    qb.at[pl.ds(q * HK, HK)], sems[q])

        def wait(j, p):
            for half_k in range(2):
                q = 2 * p + half_k
                pltpu.make_async_copy(
                    h_hbm.at[src_v.at[j, pl.ds(half_k * HK, HK)]],
                    qb.at[pl.ds(q * HK, HK)], sems[q]).wait()

        pltpu.sync_copy(z_hbm, acc_sh.at[pl.ds(sid * RPT, RPT)])
        plsc.subcore_barrier()

        for half in range(2):
            pltpu.sync_copy(src_hbm.at[wid, pl.ds(half * hh, hh)], src_v)
            pltpu.sync_copy(dst_hbm.at[wid, pl.ds(half * hh, hh)], dst_v)

            issue(0, 0)
            issue(1, 1)

            @pl.loop(0, hh // 2)
            def _(i):
                for p in range(2):
                    j = 2 * i + p
                    wait(j, p)
                    pltpu.sync_copy(qb.at[pl.ds(p * K, K)],
                                    acc_sh.at[dst_v.at[j]], add=True)

                    @pl.when(j + 2 < hh)
                    def _():
                        issue(j + 2, p)

        plsc.subcore_barrier()
        pltpu.sync_copy(acc_sh.at[pl.ds(sid * RPT, RPT)],
                        out_hbm.at[cid, pl.ds(sid * RPT, RPT)])

    return _sc_agg


BN = 512
GRID = NP // BN


def _mlp_body(h_ref, agg_ref, w1_ref, b1_ref, w2_ref, b2_ref,
              out_ref, cs_ref):
    i = pl.program_id(0)
    xb = h_ref[...] + agg_ref[0] + agg_ref[1]
    t = jnp.dot(xb, w1_ref[...], preferred_element_type=jnp.float32)
    t = jnp.maximum(t + b1_ref[...], 0.0)
    hn = jnp.dot(t, w2_ref[...], preferred_element_type=jnp.float32)
    hn = hn + b2_ref[...]
    out_ref[...] = hn
    rows = i * BN + lax.broadcasted_iota(jnp.int32, (BN, 1), 0)
    contrib = jnp.sum(jnp.where(rows < N, hn, 0.0), axis=0, keepdims=True)

    @pl.when(i == 0)
    def _():
        cs_ref[...] = contrib

    @pl.when(i != 0)
    def _():
        cs_ref[...] = cs_ref[...] + contrib


def _mlp(h, agg, W1, b1, W2, b2):
    return pl.pallas_call(
        _mlp_body,
        grid=(GRID,),
        in_specs=[
            pl.BlockSpec((BN, D), lambda i: (i, 0)),
            pl.BlockSpec((2, BN, D), lambda i: (0, i, 0)),
            pl.BlockSpec((D, D), lambda i: (0, 0)),
            pl.BlockSpec((1, D), lambda i: (0, 0)),
            pl.BlockSpec((D, D), lambda i: (0, 0)),
            pl.BlockSpec((1, D), lambda i: (0, 0)),
        ],
        out_specs=[
            pl.BlockSpec((BN, D), lambda i: (i, 0)),
            pl.BlockSpec((1, D), lambda i: (0, 0)),
        ],
        out_shape=[
            jax.ShapeDtypeStruct((NP, D), jnp.float32),
            jax.ShapeDtypeStruct((1, D), jnp.float32),
        ],
    )(h, agg, W1, b1.reshape(1, D), W2, b2.reshape(1, D))


def _final_body(h2_ref, agg_ref, w1_ref, b1_ref, w2_ref, b2_ref,
                h1_ref, nw1_ref, nb1_ref, nw2_ref, nb2_ref,
                s1_ref, s2_ref, gw1_ref, gb1_ref, gw2_ref, gb2_ref,
                nlog_ref, glog_ref, cs_ref):
    i = pl.program_id(0)
    xb = h2_ref[...] + agg_ref[0] + agg_ref[1]
    t = jnp.dot(xb, w1_ref[...], preferred_element_type=jnp.float32)
    t = jnp.maximum(t + b1_ref[...], 0.0)
    h3 = jnp.dot(t, w2_ref[...], preferred_element_type=jnp.float32)
    h3 = h3 + b2_ref[...]
    rows = i * BN + lax.broadcasted_iota(jnp.int32, (BN, 1), 0)
    contrib = jnp.sum(jnp.where(rows < N, h3, 0.0), axis=0, keepdims=True)

    @pl.when(i == 0)
    def _():
        cs_ref[...] = contrib

    @pl.when(i != 0)
    def _():
        cs_ref[...] = cs_ref[...] + contrib

    emb = jnp.concatenate([h1_ref[...], h2_ref[...], h3], axis=1)
    t2 = jnp.dot(emb, nw1_ref[...], preferred_element_type=jnp.float32)
    t2 = jnp.maximum(t2 + nb1_ref[...], 0.0)
    nlog_ref[...] = (
        jnp.dot(t2, nw2_ref[...], preferred_element_type=jnp.float32)
        + nb2_ref[...])

    @pl.when(i == GRID - 1)
    def _():
        gemb = jnp.concatenate(
            [s1_ref[...], s2_ref[...], cs_ref[...]], axis=1) * (1.0 / N)
        g = jnp.dot(gemb, gw1_ref[...], preferred_element_type=jnp.float32)
        g = jnp.maximum(g + gb1_ref[...], 0.0)
        glog_ref[...] = (
            jnp.dot(g, gw2_ref[...], preferred_element_type=jnp.float32)
            + gb2_ref[...])


def _final(h2, agg, W1, b1, W2, b2, h1, nw1, nb1, nw2, nb2,
           s1, s2, gw1, gb1, gw2, gb2):
    return pl.pallas_call(
        _final_body,
        grid=(GRID,),
        in_specs=[
            pl.BlockSpec((BN, D), lambda i: (i, 0)),
            pl.BlockSpec((2, BN, D), lambda i: (0, i, 0)),
            pl.BlockSpec((D, D), lambda i: (0, 0)),
            pl.BlockSpec((1, D), lambda i: (0, 0)),
            pl.BlockSpec((D, D), lambda i: (0, 0)),
            pl.BlockSpec((1, D), lambda i: (0, 0)),
            pl.BlockSpec((BN, D), lambda i: (i, 0)),
            pl.BlockSpec((3 * D, 4 * D), lambda i: (0, 0)),
            pl.BlockSpec((1, 4 * D), lambda i: (0, 0)),
            pl.BlockSpec((4 * D, 8), lambda i: (0, 0)),
            pl.BlockSpec((1, 8), lambda i: (0, 0)),
            pl.BlockSpec((1, D), lambda i: (0, 0)),
            pl.BlockSpec((1, D), lambda i: (0, 0)),
            pl.BlockSpec((3 * D, D), lambda i: (0, 0)),
            pl.BlockSpec((1, D), lambda i: (0, 0)),
            pl.BlockSpec((D, 8), lambda i: (0, 0)),
            pl.BlockSpec((1, 8), lambda i: (0, 0)),
        ],
        out_specs=[
            pl.BlockSpec((BN, 8), lambda i: (i, 0)),
            pl.BlockSpec((1, 8), lambda i: (0, 0)),
        ],
        out_shape=[
            jax.ShapeDtypeStruct((NP, 8), jnp.float32),
            jax.ShapeDtypeStruct((1, 8), jnp.float32),
        ],
        scratch_shapes=[pltpu.VMEM((1, D), jnp.float32)],
    )(h2, agg, W1, b1.reshape(1, D), W2, b2.reshape(1, D), h1,
      nw1, nb1, nw2, nb2, s1, s2, gw1, gb1, gw2, gb2)


def kernel(x, edge_index,
           gW1_0, gb1_0, gW2_0, gb2_0,
           gW1_1, gb1_1, gW2_1, gb2_1,
           gW1_2, gb1_2, gW2_2, gb2_2,
           gcW1, gcb1, gcW2, gcb2,
           ncW1, ncb1, ncW2, ncb2):
    f32 = jnp.float32
    src = edge_index[0]
    dst = edge_index[1]
    padv = jnp.broadcast_to(N + jnp.arange(PPW, dtype=jnp.int32), (NW, PPW))
    srcp = jnp.concatenate([src.reshape(NW, EPW), padv], axis=1)
    srcp = srcp.reshape(NW, CH, K)
    dstp = jnp.concatenate([dst.reshape(NW, EPW), padv], axis=1)
    dstp = dstp.reshape(NW, CH, K)
    zt = jnp.zeros((RPT, D), f32)

    h0 = jnp.zeros((NP, D), f32).at[:N].set(x)
    agg1 = _get_sc_agg()(h0, srcp, dstp, zt)
    h1, s1 = _mlp(h0, agg1, gW1_0, gb1_0, gW2_0, gb2_0)
    agg2 = _get_sc_agg()(h1, srcp, dstp, zt)
    h2, s2 = _mlp(h1, agg2, gW1_1, gb1_1, gW2_1, gb2_1)
    agg3 = _get_sc_agg()(h2, srcp, dstp, zt)

    w1all = ncW1.transpose(1, 0, 2).reshape(3 * D, 4 * D)
    b1all = ncb1.reshape(1, 4 * D)
    w2bd = jnp.zeros((4 * D, 8), f32)
    for c in range(4):
        w2bd = w2bd.at[c * D:(c + 1) * D, c].set(ncW2[c, :, 0])
    b2bd = jnp.zeros((1, 8), f32).at[0, :4].set(ncb2[:, 0])
    gw2p = jnp.zeros((D, 8), f32).at[:, :4].set(gcW2)
    gb2p = jnp.zeros((1, 8), f32).at[0, :4].set(gcb2)

    nlog, glog = _final(h2, agg3, gW1_2, gb1_2, gW2_2, gb2_2, h1,
                        w1all, b1all, w2bd, b2bd, s1, s2,
                        gcW1, gcb1.reshape(1, D), gw2p, gb2p)
    return glog[0, :4], nlog[:N, :4]


# speedup vs baseline: 1.0578x; 1.0578x over previous
import functools

import jax
import jax.numpy as jnp
from jax import lax
from jax.experimental import pallas as pl
from jax.experimental.pallas import tpu as pltpu
from jax.experimental.pallas import tpu_sc as plsc

N = 10000
D = 128
E = 320000
NP = 10240
K = 128
NW = 32
CH = 80
EPW = E // NW
PPW = CH * K - EPW
NSUB = 16
RPT = NP // NSUB

@functools.cache
def _get_sc_agg():
    mesh = plsc.VectorSubcoreMesh(core_axis_name="c", subcore_axis_name="s",
                                  num_cores=2, num_subcores=NSUB)

    @functools.partial(
        pl.kernel,
        out_type=jax.ShapeDtypeStruct((2, NP, D), jnp.float32),
        mesh=mesh,
        scratch_types=[
            pltpu.VMEM((CH // 2, K), jnp.int32),
            pltpu.VMEM((CH // 2, K), jnp.int32),
            pltpu.VMEM((K, D), jnp.float32),
            pltpu.VMEM((K, D), jnp.float32),
            pltpu.VMEM_SHARED((NP, D), jnp.float32),
            pltpu.SemaphoreType.DMA,
            pltpu.SemaphoreType.DMA,
        ],
    )
    def _sc_agg(h_hbm, src_hbm, dst_hbm, out_hbm, src_v, dst_v,
                buf0, buf1, acc_sh, sem0, sem1):
        cid = lax.axis_index("c")
        sid = lax.axis_index("s")
        wid = sid * 2 + cid
        hh = CH // 2
        pltpu.sync_copy(src_hbm.at[wid, pl.ds(0, hh)], src_v)
        pltpu.sync_copy(dst_hbm.at[wid, pl.ds(0, hh)], dst_v)
        pltpu.async_copy(h_hbm.at[src_v.at[0]], buf0, sem0)
        zv = jnp.zeros((16,), jnp.float32)

        @pl.loop(0, K)
        def _(r):
            for c in range(D // 16):
                buf1[r, c * 16:(c + 1) * 16] = zv

        for t in range(RPT // K):
            pltpu.sync_copy(buf1, acc_sh.at[pl.ds(sid * RPT + t * K, K)])
        plsc.subcore_barrier()

        for half in range(2):
            if half == 1:
                pltpu.sync_copy(src_hbm.at[wid, pl.ds(hh, hh)], src_v)
                pltpu.sync_copy(dst_hbm.at[wid, pl.ds(hh, hh)], dst_v)
                pltpu.async_copy(h_hbm.at[src_v.at[0]], buf0, sem0)


            @pl.loop(0, hh // 2)
            def _(i):
                j = 2 * i
                pltpu.async_copy(h_hbm.at[src_v.at[j + 1]], buf1, sem1)
                pltpu.make_async_copy(
                    h_hbm.at[src_v.at[j]], buf0, sem0).wait()
                pltpu.sync_copy(buf0, acc_sh.at[dst_v.at[j]], add=True)

                @pl.when(i < hh // 2 - 1)
                def _():
                    pltpu.async_copy(h_hbm.at[src_v.at[j + 2]], buf0, sem0)

                pltpu.make_async_copy(
                    h_hbm.at[src_v.at[j + 1]], buf1, sem1).wait()
                pltpu.sync_copy(buf1, acc_sh.at[dst_v.at[j + 1]], add=True)

        plsc.subcore_barrier()
        pltpu.sync_copy(acc_sh.at[pl.ds(sid * RPT, RPT)],
                        out_hbm.at[cid, pl.ds(sid * RPT, RPT)])

    return _sc_agg


BN = 512
GRID = NP // BN


def _mlp_body(h_ref, agg_ref, w1_ref, b1_ref, w2_ref, b2_ref,
              out_ref, cs_ref):
    i = pl.program_id(0)
    xb = h_ref[...] + agg_ref[0] + agg_ref[1]
    t = jnp.dot(xb, w1_ref[...], preferred_element_type=jnp.float32)
    t = jnp.maximum(t + b1_ref[...], 0.0)
    hn = jnp.dot(t, w2_ref[...], preferred_element_type=jnp.float32)
    hn = hn + b2_ref[...]
    out_ref[...] = hn
    rows = i * BN + lax.broadcasted_iota(jnp.int32, (BN, 1), 0)
    contrib = jnp.sum(jnp.where(rows < N, hn, 0.0), axis=0, keepdims=True)

    @pl.when(i == 0)
    def _():
        cs_ref[...] = contrib

    @pl.when(i != 0)
    def _():
        cs_ref[...] = cs_ref[...] + contrib


def _mlp(h, agg, W1, b1, W2, b2):
    return pl.pallas_call(
        _mlp_body,
        grid=(GRID,),
        in_specs=[
            pl.BlockSpec((BN, D), lambda i: (i, 0)),
            pl.BlockSpec((2, BN, D), lambda i: (0, i, 0)),
            pl.BlockSpec((D, D), lambda i: (0, 0)),
            pl.BlockSpec((1, D), lambda i: (0, 0)),
            pl.BlockSpec((D, D), lambda i: (0, 0)),
            pl.BlockSpec((1, D), lambda i: (0, 0)),
        ],
        out_specs=[
            pl.BlockSpec((BN, D), lambda i: (i, 0)),
            pl.BlockSpec((1, D), lambda i: (0, 0)),
        ],
        out_shape=[
            jax.ShapeDtypeStruct((NP, D), jnp.float32),
            jax.ShapeDtypeStruct((1, D), jnp.float32),
        ],
    )(h, agg, W1, b1.reshape(1, D), W2, b2.reshape(1, D))


def _final_body(h2_ref, agg_ref, w1_ref, b1_ref, w2_ref, b2_ref,
                h1_ref, nw1_ref, nb1_ref, nw2_ref, nb2_ref,
                s1_ref, s2_ref, gw1_ref, gb1_ref, gw2_ref, gb2_ref,
                nlog_ref, glog_ref, cs_ref):
    i = pl.program_id(0)
    xb = h2_ref[...] + agg_ref[0] + agg_ref[1]
    t = jnp.dot(xb, w1_ref[...], preferred_element_type=jnp.float32)
    t = jnp.maximum(t + b1_ref[...], 0.0)
    h3 = jnp.dot(t, w2_ref[...], preferred_element_type=jnp.float32)
    h3 = h3 + b2_ref[...]
    rows = i * BN + lax.broadcasted_iota(jnp.int32, (BN, 1), 0)
    contrib = jnp.sum(jnp.where(rows < N, h3, 0.0), axis=0, keepdims=True)

    @pl.when(i == 0)
    def _():
        cs_ref[...] = contrib

    @pl.when(i != 0)
    def _():
        cs_ref[...] = cs_ref[...] + contrib

    emb = jnp.concatenate([h1_ref[...], h2_ref[...], h3], axis=1)
    t2 = jnp.dot(emb, nw1_ref[...], preferred_element_type=jnp.float32)
    t2 = jnp.maximum(t2 + nb1_ref[...], 0.0)
    nlog_ref[...] = (
        jnp.dot(t2, nw2_ref[...], preferred_element_type=jnp.float32)
        + nb2_ref[...])

    @pl.when(i == GRID - 1)
    def _():
        gemb = jnp.concatenate(
            [s1_ref[...], s2_ref[...], cs_ref[...]], axis=1) * (1.0 / N)
        g = jnp.dot(gemb, gw1_ref[...], preferred_element_type=jnp.float32)
        g = jnp.maximum(g + gb1_ref[...], 0.0)
        glog_ref[...] = (
            jnp.dot(g, gw2_ref[...], preferred_element_type=jnp.float32)
            + gb2_ref[...])


def _final(h2, agg, W1, b1, W2, b2, h1, nw1, nb1, nw2, nb2,
           s1, s2, gw1, gb1, gw2, gb2):
    return pl.pallas_call(
        _final_body,
        grid=(GRID,),
        in_specs=[
            pl.BlockSpec((BN, D), lambda i: (i, 0)),
            pl.BlockSpec((2, BN, D), lambda i: (0, i, 0)),
            pl.BlockSpec((D, D), lambda i: (0, 0)),
            pl.BlockSpec((1, D), lambda i: (0, 0)),
            pl.BlockSpec((D, D), lambda i: (0, 0)),
            pl.BlockSpec((1, D), lambda i: (0, 0)),
            pl.BlockSpec((BN, D), lambda i: (i, 0)),
            pl.BlockSpec((3 * D, 4 * D), lambda i: (0, 0)),
            pl.BlockSpec((1, 4 * D), lambda i: (0, 0)),
            pl.BlockSpec((4 * D, 8), lambda i: (0, 0)),
            pl.BlockSpec((1, 8), lambda i: (0, 0)),
            pl.BlockSpec((1, D), lambda i: (0, 0)),
            pl.BlockSpec((1, D), lambda i: (0, 0)),
            pl.BlockSpec((3 * D, D), lambda i: (0, 0)),
            pl.BlockSpec((1, D), lambda i: (0, 0)),
            pl.BlockSpec((D, 8), lambda i: (0, 0)),
            pl.BlockSpec((1, 8), lambda i: (0, 0)),
        ],
        out_specs=[
            pl.BlockSpec((BN, 8), lambda i: (i, 0)),
            pl.BlockSpec((1, 8), lambda i: (0, 0)),
        ],
        out_shape=[
            jax.ShapeDtypeStruct((NP, 8), jnp.float32),
            jax.ShapeDtypeStruct((1, 8), jnp.float32),
        ],
        scratch_shapes=[pltpu.VMEM((1, D), jnp.float32)],
    )(h2, agg, W1, b1.reshape(1, D), W2, b2.reshape(1, D), h1,
      nw1, nb1, nw2, nb2, s1, s2, gw1, gb1, gw2, gb2)


def kernel(x, edge_index,
           gW1_0, gb1_0, gW2_0, gb2_0,
           gW1_1, gb1_1, gW2_1, gb2_1,
           gW1_2, gb1_2, gW2_2, gb2_2,
           gcW1, gcb1, gcW2, gcb2,
           ncW1, ncb1, ncW2, ncb2):
    f32 = jnp.float32
    src = edge_index[0]
    dst = edge_index[1]
    padv = jnp.broadcast_to(N + jnp.arange(PPW, dtype=jnp.int32), (NW, PPW))
    srcp = jnp.concatenate([src.reshape(NW, EPW), padv], axis=1)
    srcp = srcp.reshape(NW, CH, K)
    dstp = jnp.concatenate([dst.reshape(NW, EPW), padv], axis=1)
    dstp = dstp.reshape(NW, CH, K)

    h0 = jnp.zeros((NP, D), f32).at[:N].set(x)
    agg1 = _get_sc_agg()(h0, srcp, dstp)
    h1, s1 = _mlp(h0, agg1, gW1_0, gb1_0, gW2_0, gb2_0)
    agg2 = _get_sc_agg()(h1, srcp, dstp)
    h2, s2 = _mlp(h1, agg2, gW1_1, gb1_1, gW2_1, gb2_1)
    agg3 = _get_sc_agg()(h2, srcp, dstp)

    w1all = ncW1.transpose(1, 0, 2).reshape(3 * D, 4 * D)
    b1all = ncb1.reshape(1, 4 * D)
    w2bd = jnp.zeros((4 * D, 8), f32)
    for c in range(4):
        w2bd = w2bd.at[c * D:(c + 1) * D, c].set(ncW2[c, :, 0])
    b2bd = jnp.zeros((1, 8), f32).at[0, :4].set(ncb2[:, 0])
    gw2p = jnp.zeros((D, 8), f32).at[:, :4].set(gcW2)
    gb2p = jnp.zeros((1, 8), f32).at[0, :4].set(gcb2)

    nlog, glog = _final(h2, agg3, gW1_2, gb1_2, gW2_2, gb2_2, h1,
                        w1all, b1all, w2bd, b2bd, s1, s2,
                        gcW1, gcb1.reshape(1, D), gw2p, gb2p)
    return glog[0, :4], nlog[:N, :4]


# speedup vs baseline: 1.1029x; 1.0427x over previous
import functools

import jax
import jax.numpy as jnp
from jax import lax
from jax.experimental import pallas as pl
from jax.experimental.pallas import tpu as pltpu
from jax.experimental.pallas import tpu_sc as plsc

N = 10000
D = 128
E = 320000
NP = 10240
K = 128
NW = 32
CH = 80
EPW = E // NW
PPW = CH * K - EPW
NSUB = 16
RPT = NP // NSUB

@functools.cache
def _get_sc_agg():
    mesh = plsc.VectorSubcoreMesh(core_axis_name="c", subcore_axis_name="s",
                                  num_cores=2, num_subcores=NSUB)

    @functools.partial(
        pl.kernel,
        out_type=jax.ShapeDtypeStruct((2, NP, D), jnp.float32),
        mesh=mesh,
        scratch_types=[
            pltpu.VMEM((CH // 2, K), jnp.int32),
            pltpu.VMEM((CH // 2, K), jnp.int32),
            pltpu.VMEM((K, D), jnp.float32),
            pltpu.VMEM((K, D), jnp.float32),
            pltpu.VMEM_SHARED((NP, D), jnp.float32),
            pltpu.SemaphoreType.DMA,
            pltpu.SemaphoreType.DMA,
        ],
    )
    def _sc_agg(h_hbm, src_hbm, dst_hbm, out_hbm, src_v, dst_v,
                buf0, buf1, acc_sh, sem0, sem1):
        cid = lax.axis_index("c")
        sid = lax.axis_index("s")
        wid = sid * 2 + cid
        hh = CH // 2
        pltpu.sync_copy(src_hbm.at[wid, pl.ds(0, hh)], src_v)
        pltpu.sync_copy(dst_hbm.at[wid, pl.ds(0, hh)], dst_v)
        pltpu.async_copy(h_hbm.at[src_v.at[0]], buf0, sem0)
        zv = jnp.zeros((16,), jnp.float32)

        @pl.loop(0, K)
        def _(r):
            for c in range(D // 16):
                buf1[r, c * 16:(c + 1) * 16] = zv

        for t in range(RPT // K):
            pltpu.sync_copy(buf1, acc_sh.at[pl.ds(sid * RPT + t * K, K)])
        plsc.subcore_barrier()

        for half in range(2):
            if half == 1:
                pltpu.sync_copy(src_hbm.at[wid, pl.ds(hh, hh)], src_v)
                pltpu.sync_copy(dst_hbm.at[wid, pl.ds(hh, hh)], dst_v)
                pltpu.async_copy(h_hbm.at[src_v.at[0]], buf0, sem0)


            @pl.loop(0, hh // 2)
            def _(i):
                j = 2 * i
                pltpu.async_copy(h_hbm.at[src_v.at[j + 1]], buf1, sem1)
                pltpu.make_async_copy(
                    h_hbm.at[src_v.at[j]], buf0, sem0).wait()
                pltpu.sync_copy(buf0, acc_sh.at[dst_v.at[j]], add=True)

                @pl.when(i < hh // 2 - 1)
                def _():
                    pltpu.async_copy(h_hbm.at[src_v.at[j + 2]], buf0, sem0)

                pltpu.make_async_copy(
                    h_hbm.at[src_v.at[j + 1]], buf1, sem1).wait()
                pltpu.sync_copy(buf1, acc_sh.at[dst_v.at[j + 1]], add=True)

        plsc.subcore_barrier()
        pltpu.sync_copy(acc_sh.at[pl.ds(sid * RPT, RPT)],
                        out_hbm.at[cid, pl.ds(sid * RPT, RPT)])

    return _sc_agg


BN = 1024
GRID = NP // BN


def _mlp_body(h_ref, agg_ref, w1_ref, b1_ref, w2_ref, b2_ref,
              out_ref, cs_ref):
    i = pl.program_id(0)
    xb = h_ref[...] + agg_ref[0] + agg_ref[1]
    t = jnp.dot(xb, w1_ref[...], preferred_element_type=jnp.float32)
    t = jnp.maximum(t + b1_ref[...], 0.0)
    hn = jnp.dot(t, w2_ref[...], preferred_element_type=jnp.float32)
    hn = hn + b2_ref[...]
    out_ref[...] = hn
    rows = i * BN + lax.broadcasted_iota(jnp.int32, (BN, 1), 0)
    contrib = jnp.sum(jnp.where(rows < N, hn, 0.0), axis=0, keepdims=True)

    @pl.when(i == 0)
    def _():
        cs_ref[...] = contrib

    @pl.when(i != 0)
    def _():
        cs_ref[...] = cs_ref[...] + contrib


def _mlp(h, agg, W1, b1, W2, b2):
    return pl.pallas_call(
        _mlp_body,
        grid=(GRID,),
        in_specs=[
            pl.BlockSpec((BN, D), lambda i: (i, 0)),
            pl.BlockSpec((2, BN, D), lambda i: (0, i, 0)),
            pl.BlockSpec((D, D), lambda i: (0, 0)),
            pl.BlockSpec((1, D), lambda i: (0, 0)),
            pl.BlockSpec((D, D), lambda i: (0, 0)),
            pl.BlockSpec((1, D), lambda i: (0, 0)),
        ],
        out_specs=[
            pl.BlockSpec((BN, D), lambda i: (i, 0)),
            pl.BlockSpec((1, D), lambda i: (0, 0)),
        ],
        out_shape=[
            jax.ShapeDtypeStruct((NP, D), jnp.float32),
            jax.ShapeDtypeStruct((1, D), jnp.float32),
        ],
    )(h, agg, W1, b1.reshape(1, D), W2, b2.reshape(1, D))


def _final_body(h2_ref, agg_ref, w1_ref, b1_ref, w2_ref, b2_ref,
                h1_ref, nw1_ref, nb1_ref, nw2_ref, nb2_ref,
                s1_ref, s2_ref, gw1_ref, gb1_ref, gw2_ref, gb2_ref,
                nlog_ref, glog_ref, cs_ref):
    i = pl.program_id(0)
    xb = h2_ref[...] + agg_ref[0] + agg_ref[1]
    t = jnp.dot(xb, w1_ref[...], preferred_element_type=jnp.float32)
    t = jnp.maximum(t + b1_ref[...], 0.0)
    h3 = jnp.dot(t, w2_ref[...], preferred_element_type=jnp.float32)
    h3 = h3 + b2_ref[...]
    rows = i * BN + lax.broadcasted_iota(jnp.int32, (BN, 1), 0)
    contrib = jnp.sum(jnp.where(rows < N, h3, 0.0), axis=0, keepdims=True)

    @pl.when(i == 0)
    def _():
        cs_ref[...] = contrib

    @pl.when(i != 0)
    def _():
        cs_ref[...] = cs_ref[...] + contrib

    emb = jnp.concatenate([h1_ref[...], h2_ref[...], h3], axis=1)
    t2 = jnp.dot(emb, nw1_ref[...], preferred_element_type=jnp.float32)
    t2 = jnp.maximum(t2 + nb1_ref[...], 0.0)
    nlog_ref[...] = (
        jnp.dot(t2, nw2_ref[...], preferred_element_type=jnp.float32)
        + nb2_ref[...])

    @pl.when(i == GRID - 1)
    def _():
        gemb = jnp.concatenate(
            [s1_ref[...], s2_ref[...], cs_ref[...]], axis=1) * (1.0 / N)
        g = jnp.dot(gemb, gw1_ref[...], preferred_element_type=jnp.float32)
        g = jnp.maximum(g + gb1_ref[...], 0.0)
        glog_ref[...] = (
            jnp.dot(g, gw2_ref[...], preferred_element_type=jnp.float32)
            + gb2_ref[...])


def _final(h2, agg, W1, b1, W2, b2, h1, nw1, nb1, nw2, nb2,
           s1, s2, gw1, gb1, gw2, gb2):
    return pl.pallas_call(
        _final_body,
        grid=(GRID,),
        in_specs=[
            pl.BlockSpec((BN, D), lambda i: (i, 0)),
            pl.BlockSpec((2, BN, D), lambda i: (0, i, 0)),
            pl.BlockSpec((D, D), lambda i: (0, 0)),
            pl.BlockSpec((1, D), lambda i: (0, 0)),
            pl.BlockSpec((D, D), lambda i: (0, 0)),
            pl.BlockSpec((1, D), lambda i: (0, 0)),
            pl.BlockSpec((BN, D), lambda i: (i, 0)),
            pl.BlockSpec((3 * D, 4 * D), lambda i: (0, 0)),
            pl.BlockSpec((1, 4 * D), lambda i: (0, 0)),
            pl.BlockSpec((4 * D, 8), lambda i: (0, 0)),
            pl.BlockSpec((1, 8), lambda i: (0, 0)),
            pl.BlockSpec((1, D), lambda i: (0, 0)),
            pl.BlockSpec((1, D), lambda i: (0, 0)),
            pl.BlockSpec((3 * D, D), lambda i: (0, 0)),
            pl.BlockSpec((1, D), lambda i: (0, 0)),
            pl.BlockSpec((D, 8), lambda i: (0, 0)),
            pl.BlockSpec((1, 8), lambda i: (0, 0)),
        ],
        out_specs=[
            pl.BlockSpec((BN, 8), lambda i: (i, 0)),
            pl.BlockSpec((1, 8), lambda i: (0, 0)),
        ],
        out_shape=[
            jax.ShapeDtypeStruct((NP, 8), jnp.float32),
            jax.ShapeDtypeStruct((1, 8), jnp.float32),
        ],
        scratch_shapes=[pltpu.VMEM((1, D), jnp.float32)],
    )(h2, agg, W1, b1.reshape(1, D), W2, b2.reshape(1, D), h1,
      nw1, nb1, nw2, nb2, s1, s2, gw1, gb1, gw2, gb2)


def kernel(x, edge_index,
           gW1_0, gb1_0, gW2_0, gb2_0,
           gW1_1, gb1_1, gW2_1, gb2_1,
           gW1_2, gb1_2, gW2_2, gb2_2,
           gcW1, gcb1, gcW2, gcb2,
           ncW1, ncb1, ncW2, ncb2):
    f32 = jnp.float32
    src = edge_index[0]
    dst = edge_index[1]
    padv = jnp.broadcast_to(N + jnp.arange(PPW, dtype=jnp.int32), (NW, PPW))
    srcp = jnp.concatenate([src.reshape(NW, EPW), padv], axis=1)
    srcp = srcp.reshape(NW, CH, K)
    dstp = jnp.concatenate([dst.reshape(NW, EPW), padv], axis=1)
    dstp = dstp.reshape(NW, CH, K)

    h0 = jnp.zeros((NP, D), f32).at[:N].set(x)
    agg1 = _get_sc_agg()(h0, srcp, dstp)
    h1, s1 = _mlp(h0, agg1, gW1_0, gb1_0, gW2_0, gb2_0)
    agg2 = _get_sc_agg()(h1, srcp, dstp)
    h2, s2 = _mlp(h1, agg2, gW1_1, gb1_1, gW2_1, gb2_1)
    agg3 = _get_sc_agg()(h2, srcp, dstp)

    w1all = ncW1.transpose(1, 0, 2).reshape(3 * D, 4 * D)
    b1all = ncb1.reshape(1, 4 * D)
    w2bd = jnp.zeros((4 * D, 8), f32)
    for c in range(4):
        w2bd = w2bd.at[c * D:(c + 1) * D, c].set(ncW2[c, :, 0])
    b2bd = jnp.zeros((1, 8), f32).at[0, :4].set(ncb2[:, 0])
    gw2p = jnp.zeros((D, 8), f32).at[:, :4].set(gcW2)
    gb2p = jnp.zeros((1, 8), f32).at[0, :4].set(gcb2)

    nlog, glog = _final(h2, agg3, gW1_2, gb1_2, gW2_2, gb2_2, h1,
                        w1all, b1all, w2bd, b2bd, s1, s2,
                        gcW1, gcb1.reshape(1, D), gw2p, gb2p)
    return glog[0, :4], nlog[:N, :4]


# speedup vs baseline: 1.1239x; 1.0190x over previous
import functools

import jax
import jax.numpy as jnp
from jax import lax
from jax.experimental import pallas as pl
from jax.experimental.pallas import tpu as pltpu
from jax.experimental.pallas import tpu_sc as plsc

N = 10000
D = 128
E = 320000
NP = 10240
K = 128
NW = 32
CH = 80
EPW = E // NW
PPW = CH * K - EPW
NSUB = 16
RPT = NP // NSUB

@functools.cache
def _get_sc_agg():
    mesh = plsc.VectorSubcoreMesh(core_axis_name="c", subcore_axis_name="s",
                                  num_cores=2, num_subcores=NSUB)

    @functools.partial(
        pl.kernel,
        out_type=jax.ShapeDtypeStruct((2, NP, D), jnp.float32),
        mesh=mesh,
        scratch_types=[
            pltpu.VMEM((CH // 2, K), jnp.int32),
            pltpu.VMEM((CH // 2, K), jnp.int32),
            pltpu.VMEM((K, D), jnp.float32),
            pltpu.VMEM((K, D), jnp.float32),
            pltpu.VMEM_SHARED((NP, D), jnp.float32),
            pltpu.SemaphoreType.DMA,
            pltpu.SemaphoreType.DMA,
        ],
    )
    def _sc_agg(h_hbm, src_hbm, dst_hbm, out_hbm, src_v, dst_v,
                buf0, buf1, acc_sh, sem0, sem1):
        cid = lax.axis_index("c")
        sid = lax.axis_index("s")
        wid = sid * 2 + cid
        hh = CH // 2
        pltpu.sync_copy(src_hbm.at[wid, pl.ds(0, hh)], src_v)
        pltpu.sync_copy(dst_hbm.at[wid, pl.ds(0, hh)], dst_v)
        pltpu.async_copy(h_hbm.at[src_v.at[0]], buf0, sem0)
        zv = jnp.zeros((16,), jnp.float32)

        @pl.loop(0, K)
        def _(r):
            for c in range(D // 16):
                buf1[r, c * 16:(c + 1) * 16] = zv

        for t in range(RPT // K):
            pltpu.sync_copy(buf1, acc_sh.at[pl.ds(sid * RPT + t * K, K)])
        plsc.subcore_barrier()

        for half in range(2):
            if half == 1:
                pltpu.sync_copy(src_hbm.at[wid, pl.ds(hh, hh)], src_v)
                pltpu.sync_copy(dst_hbm.at[wid, pl.ds(hh, hh)], dst_v)
                pltpu.async_copy(h_hbm.at[src_v.at[0]], buf0, sem0)


            @pl.loop(0, hh // 2)
            def _(i):
                j = 2 * i
                pltpu.async_copy(h_hbm.at[src_v.at[j + 1]], buf1, sem1)
                pltpu.make_async_copy(
                    h_hbm.at[src_v.at[j]], buf0, sem0).wait()
                pltpu.sync_copy(buf0, acc_sh.at[dst_v.at[j]], add=True)

                @pl.when(i < hh // 2 - 1)
                def _():
                    pltpu.async_copy(h_hbm.at[src_v.at[j + 2]], buf0, sem0)

                pltpu.make_async_copy(
                    h_hbm.at[src_v.at[j + 1]], buf1, sem1).wait()
                pltpu.sync_copy(buf1, acc_sh.at[dst_v.at[j + 1]], add=True)

        plsc.subcore_barrier()
        pltpu.sync_copy(acc_sh.at[pl.ds(sid * RPT, RPT)],
                        out_hbm.at[cid, pl.ds(sid * RPT, RPT)])

    return _sc_agg


BN = 2048
GRID = NP // BN


def _mlp_body(h_ref, agg_ref, w1_ref, b1_ref, w2_ref, b2_ref,
              out_ref, cs_ref):
    i = pl.program_id(0)
    xb = h_ref[...] + agg_ref[0] + agg_ref[1]
    t = jnp.dot(xb, w1_ref[...], preferred_element_type=jnp.float32)
    t = jnp.maximum(t + b1_ref[...], 0.0)
    hn = jnp.dot(t, w2_ref[...], preferred_element_type=jnp.float32)
    hn = hn + b2_ref[...]
    out_ref[...] = hn
    rows = i * BN + lax.broadcasted_iota(jnp.int32, (BN, 1), 0)
    contrib = jnp.sum(jnp.where(rows < N, hn, 0.0), axis=0, keepdims=True)

    @pl.when(i == 0)
    def _():
        cs_ref[...] = contrib

    @pl.when(i != 0)
    def _():
        cs_ref[...] = cs_ref[...] + contrib


def _mlp(h, agg, W1, b1, W2, b2):
    return pl.pallas_call(
        _mlp_body,
        grid=(GRID,),
        in_specs=[
            pl.BlockSpec((BN, D), lambda i: (i, 0)),
            pl.BlockSpec((2, BN, D), lambda i: (0, i, 0)),
            pl.BlockSpec((D, D), lambda i: (0, 0)),
            pl.BlockSpec((1, D), lambda i: (0, 0)),
            pl.BlockSpec((D, D), lambda i: (0, 0)),
            pl.BlockSpec((1, D), lambda i: (0, 0)),
        ],
        out_specs=[
            pl.BlockSpec((BN, D), lambda i: (i, 0)),
            pl.BlockSpec((1, D), lambda i: (0, 0)),
        ],
        out_shape=[
            jax.ShapeDtypeStruct((NP, D), jnp.float32),
            jax.ShapeDtypeStruct((1, D), jnp.float32),
        ],
    )(h, agg, W1, b1.reshape(1, D), W2, b2.reshape(1, D))


def _final_body(h2_ref, agg_ref, w1_ref, b1_ref, w2_ref, b2_ref,
                h1_ref, nw1_ref, nb1_ref, nw2_ref, nb2_ref,
                s1_ref, s2_ref, gw1_ref, gb1_ref, gw2_ref, gb2_ref,
                nlog_ref, glog_ref, cs_ref):
    i = pl.program_id(0)
    xb = h2_ref[...] + agg_ref[0] + agg_ref[1]
    t = jnp.dot(xb, w1_ref[...], preferred_element_type=jnp.float32)
    t = jnp.maximum(t + b1_ref[...], 0.0)
    h3 = jnp.dot(t, w2_ref[...], preferred_element_type=jnp.float32)
    h3 = h3 + b2_ref[...]
    rows = i * BN + lax.broadcasted_iota(jnp.int32, (BN, 1), 0)
    contrib = jnp.sum(jnp.where(rows < N, h3, 0.0), axis=0, keepdims=True)

    @pl.when(i == 0)
    def _():
        cs_ref[...] = contrib

    @pl.when(i != 0)
    def _():
        cs_ref[...] = cs_ref[...] + contrib

    emb = jnp.concatenate([h1_ref[...], h2_ref[...], h3], axis=1)
    t2 = jnp.dot(emb, nw1_ref[...], preferred_element_type=jnp.float32)
    t2 = jnp.maximum(t2 + nb1_ref[...], 0.0)
    nlog_ref[...] = (
        jnp.dot(t2, nw2_ref[...], preferred_element_type=jnp.float32)
        + nb2_ref[...])

    @pl.when(i == GRID - 1)
    def _():
        gemb = jnp.concatenate(
            [s1_ref[...], s2_ref[...], cs_ref[...]], axis=1) * (1.0 / N)
        g = jnp.dot(gemb, gw1_ref[...], preferred_element_type=jnp.float32)
        g = jnp.maximum(g + gb1_ref[...], 0.0)
        glog_ref[...] = (
            jnp.dot(g, gw2_ref[...], preferred_element_type=jnp.float32)
            + gb2_ref[...])


def _final(h2, agg, W1, b1, W2, b2, h1, nw1, nb1, nw2, nb2,
           s1, s2, gw1, gb1, gw2, gb2):
    return pl.pallas_call(
        _final_body,
        grid=(GRID,),
        in_specs=[
            pl.BlockSpec((BN, D), lambda i: (i, 0)),
            pl.BlockSpec((2, BN, D), lambda i: (0, i, 0)),
            pl.BlockSpec((D, D), lambda i: (0, 0)),
            pl.BlockSpec((1, D), lambda i: (0, 0)),
            pl.BlockSpec((D, D), lambda i: (0, 0)),
            pl.BlockSpec((1, D), lambda i: (0, 0)),
            pl.BlockSpec((BN, D), lambda i: (i, 0)),
            pl.BlockSpec((3 * D, 4 * D), lambda i: (0, 0)),
            pl.BlockSpec((1, 4 * D), lambda i: (0, 0)),
            pl.BlockSpec((4 * D, 8), lambda i: (0, 0)),
            pl.BlockSpec((1, 8), lambda i: (0, 0)),
            pl.BlockSpec((1, D), lambda i: (0, 0)),
            pl.BlockSpec((1, D), lambda i: (0, 0)),
            pl.BlockSpec((3 * D, D), lambda i: (0, 0)),
            pl.BlockSpec((1, D), lambda i: (0, 0)),
            pl.BlockSpec((D, 8), lambda i: (0, 0)),
            pl.BlockSpec((1, 8), lambda i: (0, 0)),
        ],
        out_specs=[
            pl.BlockSpec((BN, 8), lambda i: (i, 0)),
            pl.BlockSpec((1, 8), lambda i: (0, 0)),
        ],
        out_shape=[
            jax.ShapeDtypeStruct((NP, 8), jnp.float32),
            jax.ShapeDtypeStruct((1, 8), jnp.float32),
        ],
        scratch_shapes=[pltpu.VMEM((1, D), jnp.float32)],
    )(h2, agg, W1, b1.reshape(1, D), W2, b2.reshape(1, D), h1,
      nw1, nb1, nw2, nb2, s1, s2, gw1, gb1, gw2, gb2)


def kernel(x, edge_index,
           gW1_0, gb1_0, gW2_0, gb2_0,
           gW1_1, gb1_1, gW2_1, gb2_1,
           gW1_2, gb1_2, gW2_2, gb2_2,
           gcW1, gcb1, gcW2, gcb2,
           ncW1, ncb1, ncW2, ncb2):
    f32 = jnp.float32
    src = edge_index[0]
    dst = edge_index[1]
    padv = jnp.broadcast_to(N + jnp.arange(PPW, dtype=jnp.int32), (NW, PPW))
    srcp = jnp.concatenate([src.reshape(NW, EPW), padv], axis=1)
    srcp = srcp.reshape(NW, CH, K)
    dstp = jnp.concatenate([dst.reshape(NW, EPW), padv], axis=1)
    dstp = dstp.reshape(NW, CH, K)

    h0 = jnp.zeros((NP, D), f32).at[:N].set(x)
    agg1 = _get_sc_agg()(h0, srcp, dstp)
    h1, s1 = _mlp(h0, agg1, gW1_0, gb1_0, gW2_0, gb2_0)
    agg2 = _get_sc_agg()(h1, srcp, dstp)
    h2, s2 = _mlp(h1, agg2, gW1_1, gb1_1, gW2_1, gb2_1)
    agg3 = _get_sc_agg()(h2, srcp, dstp)

    w1all = ncW1.transpose(1, 0, 2).reshape(3 * D, 4 * D)
    b1all = ncb1.reshape(1, 4 * D)
    w2bd = jnp.zeros((4 * D, 8), f32)
    for c in range(4):
        w2bd = w2bd.at[c * D:(c + 1) * D, c].set(ncW2[c, :, 0])
    b2bd = jnp.zeros((1, 8), f32).at[0, :4].set(ncb2[:, 0])
    gw2p = jnp.zeros((D, 8), f32).at[:, :4].set(gcW2)
    gb2p = jnp.zeros((1, 8), f32).at[0, :4].set(gcb2)

    nlog, glog = _final(h2, agg3, gW1_2, gb1_2, gW2_2, gb2_2, h1,
                        w1all, b1all, w2bd, b2bd, s1, s2,
                        gcW1, gcb1.reshape(1, D), gw2p, gb2p)
    return glog[0, :4], nlog[:N, :4]


# speedup vs baseline: 1.1399x; 1.0143x over previous
import functools

import jax
import jax.numpy as jnp
from jax import lax
from jax.experimental import pallas as pl
from jax.experimental.pallas import tpu as pltpu
from jax.experimental.pallas import tpu_sc as plsc

N = 10000
D = 128
E = 320000
NP = 10240
K = 128
NW = 32
CH = 80
EPW = E // NW
PPW = CH * K - EPW
NSUB = 16
RPT = NP // NSUB

@functools.cache
def _get_sc_agg():
    mesh = plsc.VectorSubcoreMesh(core_axis_name="c", subcore_axis_name="s",
                                  num_cores=2, num_subcores=NSUB)

    @functools.partial(
        pl.kernel,
        out_type=jax.ShapeDtypeStruct((2, NP, D), jnp.float32),
        mesh=mesh,
        scratch_types=[
            pltpu.VMEM((CH // 2, K), jnp.int32),
            pltpu.VMEM((CH // 2, K), jnp.int32),
            pltpu.VMEM((K, D), jnp.float32),
            pltpu.VMEM((K, D), jnp.float32),
            pltpu.VMEM_SHARED((NP, D), jnp.float32),
            pltpu.SemaphoreType.DMA,
            pltpu.SemaphoreType.DMA,
        ],
    )
    def _sc_agg(h_hbm, src_hbm, dst_hbm, out_hbm, src_v, dst_v,
                buf0, buf1, acc_sh, sem0, sem1):
        cid = lax.axis_index("c")
        sid = lax.axis_index("s")
        wid = sid * 2 + cid
        hh = CH // 2
        pltpu.sync_copy(src_hbm.at[wid, pl.ds(0, hh)], src_v)
        pltpu.sync_copy(dst_hbm.at[wid, pl.ds(0, hh)], dst_v)
        pltpu.async_copy(h_hbm.at[src_v.at[0]], buf0, sem0)
        zv = jnp.zeros((16,), jnp.float32)

        @pl.loop(0, K)
        def _(r):
            for c in range(D // 16):
                buf1[r, c * 16:(c + 1) * 16] = zv

        for t in range(RPT // K):
            pltpu.sync_copy(buf1, acc_sh.at[pl.ds(sid * RPT + t * K, K)])
        plsc.subcore_barrier()

        for half in range(2):
            if half == 1:
                pltpu.sync_copy(src_hbm.at[wid, pl.ds(hh, hh)], src_v)
                pltpu.sync_copy(dst_hbm.at[wid, pl.ds(hh, hh)], dst_v)
                pltpu.async_copy(h_hbm.at[src_v.at[0]], buf0, sem0)


            @pl.loop(0, hh // 2)
            def _(i):
                j = 2 * i
                pltpu.async_copy(h_hbm.at[src_v.at[j + 1]], buf1, sem1)
                pltpu.make_async_copy(
                    h_hbm.at[src_v.at[j]], buf0, sem0).wait()
                pltpu.sync_copy(buf0, acc_sh.at[dst_v.at[j]], add=True)

                @pl.when(i < hh // 2 - 1)
                def _():
                    pltpu.async_copy(h_hbm.at[src_v.at[j + 2]], buf0, sem0)

                pltpu.make_async_copy(
                    h_hbm.at[src_v.at[j + 1]], buf1, sem1).wait()
                pltpu.sync_copy(buf1, acc_sh.at[dst_v.at[j + 1]], add=True)

        plsc.subcore_barrier()
        pltpu.sync_copy(acc_sh.at[pl.ds(sid * RPT, RPT)],
                        out_hbm.at[cid, pl.ds(sid * RPT, RPT)])

    return _sc_agg


BN = 5120
GRID = NP // BN


def _mlp_body(h_ref, agg_ref, w1_ref, b1_ref, w2_ref, b2_ref,
              out_ref, cs_ref):
    i = pl.program_id(0)
    xb = h_ref[...] + agg_ref[0] + agg_ref[1]
    t = jnp.dot(xb, w1_ref[...], preferred_element_type=jnp.float32)
    t = jnp.maximum(t + b1_ref[...], 0.0)
    hn = jnp.dot(t, w2_ref[...], preferred_element_type=jnp.float32)
    hn = hn + b2_ref[...]
    out_ref[...] = hn
    rows = i * BN + lax.broadcasted_iota(jnp.int32, (BN, 1), 0)
    contrib = jnp.sum(jnp.where(rows < N, hn, 0.0), axis=0, keepdims=True)

    @pl.when(i == 0)
    def _():
        cs_ref[...] = contrib

    @pl.when(i != 0)
    def _():
        cs_ref[...] = cs_ref[...] + contrib


def _mlp(h, agg, W1, b1, W2, b2):
    return pl.pallas_call(
        _mlp_body,
        grid=(GRID,),
        in_specs=[
            pl.BlockSpec((BN, D), lambda i: (i, 0)),
            pl.BlockSpec((2, BN, D), lambda i: (0, i, 0)),
            pl.BlockSpec((D, D), lambda i: (0, 0)),
            pl.BlockSpec((1, D), lambda i: (0, 0)),
            pl.BlockSpec((D, D), lambda i: (0, 0)),
            pl.BlockSpec((1, D), lambda i: (0, 0)),
        ],
        out_specs=[
            pl.BlockSpec((BN, D), lambda i: (i, 0)),
            pl.BlockSpec((1, D), lambda i: (0, 0)),
        ],
        out_shape=[
            jax.ShapeDtypeStruct((NP, D), jnp.float32),
            jax.ShapeDtypeStruct((1, D), jnp.float32),
        ],
    )(h, agg, W1, b1.reshape(1, D), W2, b2.reshape(1, D))


def _final_body(h2_ref, agg_ref, w1_ref, b1_ref, w2_ref, b2_ref,
                h1_ref, nw1_ref, nb1_ref, nw2_ref, nb2_ref,
                s1_ref, s2_ref, gw1_ref, gb1_ref, gw2_ref, gb2_ref,
                nlog_ref, glog_ref, cs_ref):
    i = pl.program_id(0)
    xb = h2_ref[...] + agg_ref[0] + agg_ref[1]
    t = jnp.dot(xb, w1_ref[...], preferred_element_type=jnp.float32)
    t = jnp.maximum(t + b1_ref[...], 0.0)
    h3 = jnp.dot(t, w2_ref[...], preferred_element_type=jnp.float32)
    h3 = h3 + b2_ref[...]
    rows = i * BN + lax.broadcasted_iota(jnp.int32, (BN, 1), 0)
    contrib = jnp.sum(jnp.where(rows < N, h3, 0.0), axis=0, keepdims=True)

    @pl.when(i == 0)
    def _():
        cs_ref[...] = contrib

    @pl.when(i != 0)
    def _():
        cs_ref[...] = cs_ref[...] + contrib

    emb = jnp.concatenate([h1_ref[...], h2_ref[...], h3], axis=1)
    t2 = jnp.dot(emb, nw1_ref[...], preferred_element_type=jnp.float32)
    t2 = jnp.maximum(t2 + nb1_ref[...], 0.0)
    nlog_ref[...] = (
        jnp.dot(t2, nw2_ref[...], preferred_element_type=jnp.float32)
        + nb2_ref[...])

    @pl.when(i == GRID - 1)
    def _():
        gemb = jnp.concatenate(
            [s1_ref[...], s2_ref[...], cs_ref[...]], axis=1) * (1.0 / N)
        g = jnp.dot(gemb, gw1_ref[...], preferred_element_type=jnp.float32)
        g = jnp.maximum(g + gb1_ref[...], 0.0)
        glog_ref[...] = (
            jnp.dot(g, gw2_ref[...], preferred_element_type=jnp.float32)
            + gb2_ref[...])


def _final(h2, agg, W1, b1, W2, b2, h1, nw1, nb1, nw2, nb2,
           s1, s2, gw1, gb1, gw2, gb2):
    return pl.pallas_call(
        _final_body,
        grid=(GRID,),
        in_specs=[
            pl.BlockSpec((BN, D), lambda i: (i, 0)),
            pl.BlockSpec((2, BN, D), lambda i: (0, i, 0)),
            pl.BlockSpec((D, D), lambda i: (0, 0)),
            pl.BlockSpec((1, D), lambda i: (0, 0)),
            pl.BlockSpec((D, D), lambda i: (0, 0)),
            pl.BlockSpec((1, D), lambda i: (0, 0)),
            pl.BlockSpec((BN, D), lambda i: (i, 0)),
            pl.BlockSpec((3 * D, 4 * D), lambda i: (0, 0)),
            pl.BlockSpec((1, 4 * D), lambda i: (0, 0)),
            pl.BlockSpec((4 * D, 8), lambda i: (0, 0)),
            pl.BlockSpec((1, 8), lambda i: (0, 0)),
            pl.BlockSpec((1, D), lambda i: (0, 0)),
            pl.BlockSpec((1, D), lambda i: (0, 0)),
            pl.BlockSpec((3 * D, D), lambda i: (0, 0)),
            pl.BlockSpec((1, D), lambda i: (0, 0)),
            pl.BlockSpec((D, 8), lambda i: (0, 0)),
            pl.BlockSpec((1, 8), lambda i: (0, 0)),
        ],
        out_specs=[
            pl.BlockSpec((BN, 8), lambda i: (i, 0)),
            pl.BlockSpec((1, 8), lambda i: (0, 0)),
        ],
        out_shape=[
            jax.ShapeDtypeStruct((NP, 8), jnp.float32),
            jax.ShapeDtypeStruct((1, 8), jnp.float32),
        ],
        scratch_shapes=[pltpu.VMEM((1, D), jnp.float32)],
    )(h2, agg, W1, b1.reshape(1, D), W2, b2.reshape(1, D), h1,
      nw1, nb1, nw2, nb2, s1, s2, gw1, gb1, gw2, gb2)


def kernel(x, edge_index,
           gW1_0, gb1_0, gW2_0, gb2_0,
           gW1_1, gb1_1, gW2_1, gb2_1,
           gW1_2, gb1_2, gW2_2, gb2_2,
           gcW1, gcb1, gcW2, gcb2,
           ncW1, ncb1, ncW2, ncb2):
    f32 = jnp.float32
    src = edge_index[0]
    dst = edge_index[1]
    padv = jnp.broadcast_to(N + jnp.arange(PPW, dtype=jnp.int32), (NW, PPW))
    srcp = jnp.concatenate([src.reshape(NW, EPW), padv], axis=1)
    srcp = srcp.reshape(NW, CH, K)
    dstp = jnp.concatenate([dst.reshape(NW, EPW), padv], axis=1)
    dstp = dstp.reshape(NW, CH, K)

    h0 = jnp.zeros((NP, D), f32).at[:N].set(x)
    agg1 = _get_sc_agg()(h0, srcp, dstp)
    h1, s1 = _mlp(h0, agg1, gW1_0, gb1_0, gW2_0, gb2_0)
    agg2 = _get_sc_agg()(h1, srcp, dstp)
    h2, s2 = _mlp(h1, agg2, gW1_1, gb1_1, gW2_1, gb2_1)
    agg3 = _get_sc_agg()(h2, srcp, dstp)

    w1all = ncW1.transpose(1, 0, 2).reshape(3 * D, 4 * D)
    b1all = ncb1.reshape(1, 4 * D)
    w2bd = jnp.zeros((4 * D, 8), f32)
    for c in range(4):
        w2bd = w2bd.at[c * D:(c + 1) * D, c].set(ncW2[c, :, 0])
    b2bd = jnp.zeros((1, 8), f32).at[0, :4].set(ncb2[:, 0])
    gw2p = jnp.zeros((D, 8), f32).at[:, :4].set(gcW2)
    gb2p = jnp.zeros((1, 8), f32).at[0, :4].set(gcb2)

    nlog, glog = _final(h2, agg3, gW1_2, gb1_2, gW2_2, gb2_2, h1,
                        w1all, b1all, w2bd, b2bd, s1, s2,
                        gcW1, gcb1.reshape(1, D), gw2p, gb2p)
    return glog[0, :4], nlog[:N, :4]
